# baseline (speedup 1.0000x reference)
import functools

import jax
import jax.numpy as jnp
from jax.experimental import pallas as pl

TOP_K = 8
NUM_EXPERTS = 64
BLOCK_TOKENS = 1024


def _topk_kernel(x_ref, w_ref, id_ref):
    x = x_ref[:, :]
    m = jnp.max(x, axis=1, keepdims=True)
    e = jnp.exp(x - m)
    iota = jax.lax.broadcasted_iota(jnp.int32, e.shape, 1)
    ws = []
    ids = []
    for _ in range(TOP_K):
        cur = jnp.max(e, axis=1, keepdims=True)
        hit = e == cur
        idx = jnp.min(jnp.where(hit, iota, NUM_EXPERTS), axis=1, keepdims=True)
        ws.append(cur)
        ids.append(idx)
        e = jnp.where(iota == idx, -1.0, e)
    w = jnp.concatenate(ws, axis=1)
    w = w / jnp.sum(w, axis=1, keepdims=True)
    w_ref[:, :] = w
    id_ref[:, :] = jnp.concatenate(ids, axis=1)


def kernel(router_logits_fp32, topk_ids, topk_weights):
    del topk_ids, topk_weights
    n, _ = router_logits_fp32.shape
    grid = (n // BLOCK_TOKENS,)
    w, ids = pl.pallas_call(
        _topk_kernel,
        grid=grid,
        in_specs=[pl.BlockSpec((BLOCK_TOKENS, NUM_EXPERTS), lambda i: (i, 0))],
        out_specs=[
            pl.BlockSpec((BLOCK_TOKENS, TOP_K), lambda i: (i, 0)),
            pl.BlockSpec((BLOCK_TOKENS, TOP_K), lambda i: (i, 0)),
        ],
        out_shape=[
            jax.ShapeDtypeStruct((n, TOP_K), jnp.float32),
            jax.ShapeDtypeStruct((n, TOP_K), jnp.int32),
        ],
    )(router_logits_fp32)
    return (w, ids, ids)


# speedup vs baseline: 1.3566x; 1.3566x over previous
import jax
import jax.numpy as jnp
from jax.experimental import pallas as pl
from jax.experimental.pallas import tpu as pltpu

TOP_K = 8
NUM_EXPERTS = 64
BLOCK_TOKENS = 2048
NEG_INF = float("-inf")


def _topk_kernel(x_ref, w_ref, id_ref):
    e = x_ref[:, :]
    iota_f = jax.lax.broadcasted_iota(jnp.int32, e.shape, 1).astype(jnp.float32)
    ws = []
    ids = []
    for _ in range(TOP_K):
        cur = jnp.max(e, axis=1, keepdims=True)
        hit = e == cur
        idx = jnp.min(jnp.where(hit, iota_f, 64.0), axis=1, keepdims=True)
        ws.append(cur)
        ids.append(idx)
        e = jnp.where(iota_f == idx, NEG_INF, e)
    w = jnp.concatenate(ws, axis=1)
    w = jnp.exp(w - w[:, :1])
    w = w / jnp.sum(w, axis=1, keepdims=True)
    w_ref[:, :] = w
    id_ref[:, :] = jnp.concatenate(ids, axis=1).astype(jnp.int32)


def kernel(router_logits_fp32, topk_ids, topk_weights):
    del topk_ids, topk_weights
    n, _ = router_logits_fp32.shape
    grid = (n // BLOCK_TOKENS,)
    w, ids = pl.pallas_call(
        _topk_kernel,
        grid=grid,
        in_specs=[pl.BlockSpec((BLOCK_TOKENS, NUM_EXPERTS), lambda i: (i, 0))],
        out_specs=[
            pl.BlockSpec((BLOCK_TOKENS, TOP_K), lambda i: (i, 0)),
            pl.BlockSpec((BLOCK_TOKENS, TOP_K), lambda i: (i, 0)),
        ],
        out_shape=[
            jax.ShapeDtypeStruct((n, TOP_K), jnp.float32),
            jax.ShapeDtypeStruct((n, TOP_K), jnp.int32),
        ],
        compiler_params=pltpu.CompilerParams(
            dimension_semantics=("parallel",),
        ),
    )(router_logits_fp32)
    return (w, ids, ids)


# speedup vs baseline: 2.4567x; 1.8109x over previous
import jax
import jax.numpy as jnp
from jax.experimental import pallas as pl
from jax.experimental.pallas import tpu as pltpu

TOP_K = 8
NUM_EXPERTS = 64
BLOCK_TOKENS = 2048
NEG_INF = float("-inf")


def _topk_kernel(x_ref, w_ref, id_ref):
    xt = x_ref[:, :].T
    iota_f = jax.lax.broadcasted_iota(jnp.int32, xt.shape, 0).astype(jnp.float32)
    ws = []
    ids = []
    for _ in range(TOP_K):
        cur = jnp.max(xt, axis=0, keepdims=True)
        hit = xt == cur
        idx = jnp.min(jnp.where(hit, iota_f, 64.0), axis=0, keepdims=True)
        ws.append(cur)
        ids.append(idx)
        xt = jnp.where(iota_f == idx, NEG_INF, xt)
    w = jnp.concatenate(ws, axis=0)
    w = jnp.exp(w - w[:1, :])
    w = w / jnp.sum(w, axis=0, keepdims=True)
    w_ref[:, :] = w.T
    id_ref[:, :] = jnp.concatenate(ids, axis=0).T.astype(jnp.int32)


def kernel(router_logits_fp32, topk_ids, topk_weights):
    del topk_ids, topk_weights
    n, _ = router_logits_fp32.shape
    grid = (n // BLOCK_TOKENS,)
    w, ids = pl.pallas_call(
        _topk_kernel,
        grid=grid,
        in_specs=[pl.BlockSpec((BLOCK_TOKENS, NUM_EXPERTS), lambda i: (i, 0))],
        out_specs=[
            pl.BlockSpec((BLOCK_TOKENS, TOP_K), lambda i: (i, 0)),
            pl.BlockSpec((BLOCK_TOKENS, TOP_K), lambda i: (i, 0)),
        ],
        out_shape=[
            jax.ShapeDtypeStruct((n, TOP_K), jnp.float32),
            jax.ShapeDtypeStruct((n, TOP_K), jnp.int32),
        ],
        compiler_params=pltpu.CompilerParams(
            dimension_semantics=("parallel",),
        ),
    )(router_logits_fp32)
    return (w, ids, ids)
